# baseline (speedup 1.0000x reference)
import functools

import jax
import jax.numpy as jnp
from jax import lax
from jax.experimental import pallas as pl
from jax.experimental.pallas import tpu as pltpu
from jax.experimental.pallas import tpu_sc as plsc

N = 10000
E = 320000
D = 128

NC = 2
NS = 16
NW = NC * NS
KB = 128
KN = 80
NBUF = 2
EP = NW * KN * KB
NP = 10240
RS = NP // NS

_mesh = plsc.VectorSubcoreMesh(core_axis_name="c", subcore_axis_name="s")


@functools.partial(
    pl.kernel,
    out_type=(
        jax.ShapeDtypeStruct((NC, NP, D), jnp.float32),
        jax.ShapeDtypeStruct((NC, NP), jnp.float32),
    ),
    mesh=_mesh,
    scratch_types=[
        [pltpu.VMEM((2, KB), jnp.int32) for _ in range(NBUF)],
        [pltpu.VMEM((KB, D), jnp.float32) for _ in range(NBUF)],
        pltpu.VMEM((KB,), jnp.float32),
        pltpu.VMEM_SHARED((NP, D), jnp.float32),
        pltpu.VMEM_SHARED((NP,), jnp.float32),
        [pltpu.SemaphoreType.DMA for _ in range(NBUF)],
        [pltpu.SemaphoreType.DMA for _ in range(NBUF)],
        [pltpu.SemaphoreType.DMA for _ in range(NBUF)],
        [pltpu.SemaphoreType.DMA for _ in range(NBUF)],
    ],
)
def _sc_segsum(y_hbm, eidx_hbm, znd_hbm, zn_hbm,
               acc_out, cnt_out,
               idx, rows, ones_v, acc_sh, cnt_sh,
               isems, gsems, ssems, csems):
    cid = lax.axis_index("c")
    sid = lax.axis_index("s")
    wid = sid * NC + cid

    for i in range(KB // 16):
        ones_v[pl.ds(i * 16, 16)] = jnp.full((16,), 1.0, jnp.float32)

    pltpu.sync_copy(znd_hbm.at[pl.ds(sid * RS, RS)],
                    acc_sh.at[pl.ds(sid * RS, RS)])

    @pl.when(sid == 0)
    def _():
        pltpu.sync_copy(zn_hbm, cnt_sh)

    plsc.subcore_barrier()

    def fire_idx(j, b):
        pltpu.async_copy(eidx_hbm.at[wid, j], idx[b], isems[b])

    def wait_idx(j, b):
        pltpu.make_async_copy(eidx_hbm.at[wid, j], idx[b], isems[b]).wait()

    def fire_gather(b):
        pltpu.async_copy(y_hbm.at[idx[b].at[0]], rows[b], gsems[b])

    def wait_gather(b):
        pltpu.make_async_copy(y_hbm.at[idx[b].at[0]], rows[b],
                              gsems[b]).wait()

    def fire_scatter(b):
        return [pltpu.async_copy(rows[b], acc_sh.at[idx[b].at[1]],
                                 ssems[b], add=True),
                pltpu.async_copy(ones_v, cnt_sh.at[idx[b].at[1]],
                                 csems[b], add=True)]

    for b in range(NBUF):
        fire_idx(b, b)
    for b in range(NBUF):
        wait_idx(b, b)
        fire_gather(b)

    def group(g, carry):
        sdescs = []
        for b in range(NBUF):
            wait_gather(b)
            sdescs.append(fire_scatter(b))
        for b in range(NBUF):
            for dsc in sdescs[b]:
                dsc.wait()
            fire_idx((g + 1) * NBUF + b, b)
        for b in range(NBUF):
            wait_idx((g + 1) * NBUF + b, b)
            fire_gather(b)
        return carry

    lax.fori_loop(0, KN // NBUF - 1, group, 0)

    last = []
    for b in range(NBUF):
        wait_gather(b)
        last.append(fire_scatter(b))
    for descs in last:
        for dsc in descs:
            dsc.wait()

    plsc.subcore_barrier()

    pltpu.sync_copy(acc_sh.at[pl.ds(sid * RS, RS)],
                    acc_out.at[cid, pl.ds(sid * RS, RS)])

    @pl.when(sid == 0)
    def _():
        pltpu.sync_copy(cnt_sh, cnt_out.at[cid])


BR = 1000


def _pre_body(x_ref, wl_ref, wr_ref, b_ref, y_ref, r_ref):
    xb = x_ref[...]
    y_ref[...] = jnp.dot(xb, wl_ref[...], preferred_element_type=jnp.float32)
    r_ref[...] = (jnp.dot(xb, wr_ref[...], preferred_element_type=jnp.float32)
                  + b_ref[...])


_pre = pl.pallas_call(
    _pre_body,
    grid=(N // BR,),
    in_specs=[
        pl.BlockSpec((BR, D), lambda i: (i, 0)),
        pl.BlockSpec((D, D), lambda i: (0, 0)),
        pl.BlockSpec((D, D), lambda i: (0, 0)),
        pl.BlockSpec((1, D), lambda i: (0, 0)),
    ],
    out_specs=[pl.BlockSpec((BR, D), lambda i: (i, 0)),
               pl.BlockSpec((BR, D), lambda i: (i, 0))],
    out_shape=[jax.ShapeDtypeStruct((N, D), jnp.float32)] * 2,
)


def _fin_body(agg_ref, cnt_ref, r_ref, o_ref):
    agg = agg_ref[0] + agg_ref[1]
    cnt = cnt_ref[...]
    inv = 1.0 / jnp.maximum(cnt[:, 0:1] + cnt[:, 1:2], 1.0)
    o_ref[...] = jnp.maximum(agg * inv + r_ref[...], 0.0)


_fin = pl.pallas_call(
    _fin_body,
    grid=(N // BR,),
    in_specs=[
        pl.BlockSpec((NC, BR, D), lambda i: (0, i, 0)),
        pl.BlockSpec((BR, NC), lambda i: (i, 0)),
        pl.BlockSpec((BR, D), lambda i: (i, 0)),
    ],
    out_specs=pl.BlockSpec((BR, D), lambda i: (i, 0)),
    out_shape=jax.ShapeDtypeStruct((N, D), jnp.float32),
)


def kernel(x, edge_index, Wl1, Wr1, b1, Wl2, Wr2, b2):
    ei = edge_index.astype(jnp.int32)
    pad = EP - E
    ar = jnp.arange(pad, dtype=jnp.int32)
    src = jnp.concatenate([ei[0], ar % N])
    dst = jnp.concatenate([ei[1], N + (ar % (NP - N))])
    eidx = jnp.stack([src.reshape(NW, KN, KB), dst.reshape(NW, KN, KB)],
                     axis=2)
    znd = jnp.zeros((NP, D), jnp.float32)
    zn = jnp.zeros((NP,), jnp.float32)
    wls = jnp.stack([Wl1, Wl2])
    wrs = jnp.stack([Wr1, Wr2])
    bs = jnp.stack([b1.reshape(1, D), b2.reshape(1, D)])

    def layer(i, h):
        wl = lax.dynamic_index_in_dim(wls, i, keepdims=False)
        wr = lax.dynamic_index_in_dim(wrs, i, keepdims=False)
        b = lax.dynamic_index_in_dim(bs, i, keepdims=False)
        y, r = _pre(h, wl, wr, b)
        agg, cnt = _sc_segsum(y, eidx, znd, zn)
        return _fin(agg, cnt.T, r)

    return lax.fori_loop(0, 2, layer, x)


# speedup vs baseline: 1.0535x; 1.0535x over previous
import functools

import jax
import jax.numpy as jnp
from jax import lax
from jax.experimental import pallas as pl
from jax.experimental.pallas import tpu as pltpu
from jax.experimental.pallas import tpu_sc as plsc

N = 10000
E = 320000
D = 128

NC = 2
NS = 16
NW = NC * NS
KB = 80
KN = 128
NBUF = 4
EP = NW * KN * KB
NP = 10240
RS = NP // NS

_mesh = plsc.VectorSubcoreMesh(core_axis_name="c", subcore_axis_name="s")


@functools.partial(
    pl.kernel,
    out_type=(
        jax.ShapeDtypeStruct((NC, NP, D), jnp.float32),
        jax.ShapeDtypeStruct((NC, NP), jnp.float32),
    ),
    mesh=_mesh,
    scratch_types=[
        [pltpu.VMEM((2, KB), jnp.int32) for _ in range(NBUF)],
        [pltpu.VMEM((KB, D), jnp.float32) for _ in range(NBUF)],
        pltpu.VMEM((KB,), jnp.float32),
        pltpu.VMEM_SHARED((NP, D), jnp.float32),
        pltpu.VMEM_SHARED((NP,), jnp.float32),
        [pltpu.SemaphoreType.DMA for _ in range(NBUF)],
        [pltpu.SemaphoreType.DMA for _ in range(NBUF)],
        [pltpu.SemaphoreType.DMA for _ in range(NBUF)],
        [pltpu.SemaphoreType.DMA for _ in range(NBUF)],
    ],
)
def _sc_segsum(y_hbm, eidx_hbm, znd_hbm, zn_hbm,
               acc_out, cnt_out,
               idx, rows, ones_v, acc_sh, cnt_sh,
               isems, gsems, ssems, csems):
    cid = lax.axis_index("c")
    sid = lax.axis_index("s")
    wid = sid * NC + cid

    for i in range(KB // 16):
        ones_v[pl.ds(i * 16, 16)] = jnp.full((16,), 1.0, jnp.float32)

    pltpu.sync_copy(znd_hbm.at[pl.ds(sid * RS, RS)],
                    acc_sh.at[pl.ds(sid * RS, RS)])

    @pl.when(sid == 0)
    def _():
        pltpu.sync_copy(zn_hbm, cnt_sh)

    plsc.subcore_barrier()

    def fire_idx(j, b):
        pltpu.async_copy(eidx_hbm.at[wid, j], idx[b], isems[b])

    def wait_idx(j, b):
        pltpu.make_async_copy(eidx_hbm.at[wid, j], idx[b], isems[b]).wait()

    def fire_gather(b):
        pltpu.async_copy(y_hbm.at[idx[b].at[0]], rows[b], gsems[b])

    def wait_gather(b):
        pltpu.make_async_copy(y_hbm.at[idx[b].at[0]], rows[b],
                              gsems[b]).wait()

    def fire_scatter(b):
        return [pltpu.async_copy(rows[b], acc_sh.at[idx[b].at[1]],
                                 ssems[b], add=True),
                pltpu.async_copy(ones_v, cnt_sh.at[idx[b].at[1]],
                                 csems[b], add=True)]

    for b in range(NBUF):
        fire_idx(b, b)
    for b in range(NBUF):
        wait_idx(b, b)
        fire_gather(b)

    def group(g, carry):
        sdescs = []
        for b in range(NBUF):
            wait_gather(b)
            sdescs.append(fire_scatter(b))
        for b in range(NBUF):
            for dsc in sdescs[b]:
                dsc.wait()
            fire_idx((g + 1) * NBUF + b, b)
        for b in range(NBUF):
            wait_idx((g + 1) * NBUF + b, b)
            fire_gather(b)
        return carry

    lax.fori_loop(0, KN // NBUF - 1, group, 0)

    last = []
    for b in range(NBUF):
        wait_gather(b)
        last.append(fire_scatter(b))
    for descs in last:
        for dsc in descs:
            dsc.wait()

    plsc.subcore_barrier()

    pltpu.sync_copy(acc_sh.at[pl.ds(sid * RS, RS)],
                    acc_out.at[cid, pl.ds(sid * RS, RS)])

    @pl.when(sid == 0)
    def _():
        pltpu.sync_copy(cnt_sh, cnt_out.at[cid])


BR = 1000


def _pre_body(x_ref, wl_ref, wr_ref, b_ref, y_ref, r_ref):
    xb = x_ref[...]
    y_ref[...] = jnp.dot(xb, wl_ref[...], preferred_element_type=jnp.float32)
    r_ref[...] = (jnp.dot(xb, wr_ref[...], preferred_element_type=jnp.float32)
                  + b_ref[...])


_pre = pl.pallas_call(
    _pre_body,
    grid=(N // BR,),
    in_specs=[
        pl.BlockSpec((BR, D), lambda i: (i, 0)),
        pl.BlockSpec((D, D), lambda i: (0, 0)),
        pl.BlockSpec((D, D), lambda i: (0, 0)),
        pl.BlockSpec((1, D), lambda i: (0, 0)),
    ],
    out_specs=[pl.BlockSpec((BR, D), lambda i: (i, 0)),
               pl.BlockSpec((BR, D), lambda i: (i, 0))],
    out_shape=[jax.ShapeDtypeStruct((N, D), jnp.float32)] * 2,
)


def _fin_body(agg_ref, cnt_ref, r_ref, o_ref):
    agg = agg_ref[0] + agg_ref[1]
    cnt = cnt_ref[...]
    inv = 1.0 / jnp.maximum(cnt[:, 0:1] + cnt[:, 1:2], 1.0)
    o_ref[...] = jnp.maximum(agg * inv + r_ref[...], 0.0)


_fin = pl.pallas_call(
    _fin_body,
    grid=(N // BR,),
    in_specs=[
        pl.BlockSpec((NC, BR, D), lambda i: (0, i, 0)),
        pl.BlockSpec((BR, NC), lambda i: (i, 0)),
        pl.BlockSpec((BR, D), lambda i: (i, 0)),
    ],
    out_specs=pl.BlockSpec((BR, D), lambda i: (i, 0)),
    out_shape=jax.ShapeDtypeStruct((N, D), jnp.float32),
)


def kernel(x, edge_index, Wl1, Wr1, b1, Wl2, Wr2, b2):
    ei = edge_index.astype(jnp.int32)
    pad = EP - E
    ar = jnp.arange(pad, dtype=jnp.int32)
    src = jnp.concatenate([ei[0], ar % N])
    dst = jnp.concatenate([ei[1], N + (ar % (NP - N))])
    eidx = jnp.stack([src.reshape(NW, KN, KB), dst.reshape(NW, KN, KB)],
                     axis=2)
    znd = jnp.zeros((NP, D), jnp.float32)
    zn = jnp.zeros((NP,), jnp.float32)
    wls = jnp.stack([Wl1, Wl2])
    wrs = jnp.stack([Wr1, Wr2])
    bs = jnp.stack([b1.reshape(1, D), b2.reshape(1, D)])

    def layer(i, h):
        wl = lax.dynamic_index_in_dim(wls, i, keepdims=False)
        wr = lax.dynamic_index_in_dim(wrs, i, keepdims=False)
        b = lax.dynamic_index_in_dim(bs, i, keepdims=False)
        y, r = _pre(h, wl, wr, b)
        agg, cnt = _sc_segsum(y, eidx, znd, zn)
        return _fin(agg, cnt.T, r)

    return lax.fori_loop(0, 2, layer, x)


# speedup vs baseline: 1.0550x; 1.0014x over previous
import functools

import jax
import jax.numpy as jnp
from jax import lax
from jax.experimental import pallas as pl
from jax.experimental.pallas import tpu as pltpu
from jax.experimental.pallas import tpu_sc as plsc

N = 10000
E = 320000
D = 128

NC = 2
NS = 16
NW = NC * NS
KB = 64
KN = 160
NBUF = 5
EP = NW * KN * KB
NP = 10240
RS = NP // NS

_mesh = plsc.VectorSubcoreMesh(core_axis_name="c", subcore_axis_name="s")


@functools.partial(
    pl.kernel,
    out_type=(
        jax.ShapeDtypeStruct((NC, NP, D), jnp.float32),
        jax.ShapeDtypeStruct((NC, NP), jnp.float32),
    ),
    mesh=_mesh,
    scratch_types=[
        [pltpu.VMEM((2, KB), jnp.int32) for _ in range(NBUF)],
        [pltpu.VMEM((KB, D), jnp.float32) for _ in range(NBUF)],
        pltpu.VMEM((KB,), jnp.float32),
        pltpu.VMEM_SHARED((NP, D), jnp.float32),
        pltpu.VMEM_SHARED((NP,), jnp.float32),
        [pltpu.SemaphoreType.DMA for _ in range(NBUF)],
        [pltpu.SemaphoreType.DMA for _ in range(NBUF)],
        [pltpu.SemaphoreType.DMA for _ in range(NBUF)],
        [pltpu.SemaphoreType.DMA for _ in range(NBUF)],
    ],
)
def _sc_segsum(y_hbm, eidx_hbm, znd_hbm, zn_hbm,
               acc_out, cnt_out,
               idx, rows, ones_v, acc_sh, cnt_sh,
               isems, gsems, ssems, csems):
    cid = lax.axis_index("c")
    sid = lax.axis_index("s")
    wid = sid * NC + cid

    for i in range(KB // 16):
        ones_v[pl.ds(i * 16, 16)] = jnp.full((16,), 1.0, jnp.float32)

    pltpu.sync_copy(znd_hbm.at[pl.ds(sid * RS, RS)],
                    acc_sh.at[pl.ds(sid * RS, RS)])

    @pl.when(sid == 0)
    def _():
        pltpu.sync_copy(zn_hbm, cnt_sh)

    plsc.subcore_barrier()

    def fire_idx(j, b):
        pltpu.async_copy(eidx_hbm.at[wid, j], idx[b], isems[b])

    def wait_idx(j, b):
        pltpu.make_async_copy(eidx_hbm.at[wid, j], idx[b], isems[b]).wait()

    def fire_gather(b):
        pltpu.async_copy(y_hbm.at[idx[b].at[0]], rows[b], gsems[b])

    def wait_gather(b):
        pltpu.make_async_copy(y_hbm.at[idx[b].at[0]], rows[b],
                              gsems[b]).wait()

    def fire_scatter(b):
        return [pltpu.async_copy(rows[b], acc_sh.at[idx[b].at[1]],
                                 ssems[b], add=True),
                pltpu.async_copy(ones_v, cnt_sh.at[idx[b].at[1]],
                                 csems[b], add=True)]

    for b in range(NBUF):
        fire_idx(b, b)
    for b in range(NBUF):
        wait_idx(b, b)
        fire_gather(b)

    def group(g, carry):
        sdescs = []
        for b in range(NBUF):
            wait_gather(b)
            sdescs.append(fire_scatter(b))
        for b in range(NBUF):
            for dsc in sdescs[b]:
                dsc.wait()
            fire_idx((g + 1) * NBUF + b, b)
        for b in range(NBUF):
            wait_idx((g + 1) * NBUF + b, b)
            fire_gather(b)
        return carry

    lax.fori_loop(0, KN // NBUF - 1, group, 0)

    last = []
    for b in range(NBUF):
        wait_gather(b)
        last.append(fire_scatter(b))
    for descs in last:
        for dsc in descs:
            dsc.wait()

    plsc.subcore_barrier()

    pltpu.sync_copy(acc_sh.at[pl.ds(sid * RS, RS)],
                    acc_out.at[cid, pl.ds(sid * RS, RS)])

    @pl.when(sid == 0)
    def _():
        pltpu.sync_copy(cnt_sh, cnt_out.at[cid])


BR = 1000


def _pre_body(x_ref, wl_ref, wr_ref, b_ref, y_ref, r_ref):
    xb = x_ref[...]
    y_ref[...] = jnp.dot(xb, wl_ref[...], preferred_element_type=jnp.float32)
    r_ref[...] = (jnp.dot(xb, wr_ref[...], preferred_element_type=jnp.float32)
                  + b_ref[...])


_pre = pl.pallas_call(
    _pre_body,
    grid=(N // BR,),
    in_specs=[
        pl.BlockSpec((BR, D), lambda i: (i, 0)),
        pl.BlockSpec((D, D), lambda i: (0, 0)),
        pl.BlockSpec((D, D), lambda i: (0, 0)),
        pl.BlockSpec((1, D), lambda i: (0, 0)),
    ],
    out_specs=[pl.BlockSpec((BR, D), lambda i: (i, 0)),
               pl.BlockSpec((BR, D), lambda i: (i, 0))],
    out_shape=[jax.ShapeDtypeStruct((N, D), jnp.float32)] * 2,
)


def _fin_body(agg_ref, cnt_ref, r_ref, o_ref):
    agg = agg_ref[0] + agg_ref[1]
    cnt = cnt_ref[...]
    inv = 1.0 / jnp.maximum(cnt[:, 0:1] + cnt[:, 1:2], 1.0)
    o_ref[...] = jnp.maximum(agg * inv + r_ref[...], 0.0)


_fin = pl.pallas_call(
    _fin_body,
    grid=(N // BR,),
    in_specs=[
        pl.BlockSpec((NC, BR, D), lambda i: (0, i, 0)),
        pl.BlockSpec((BR, NC), lambda i: (i, 0)),
        pl.BlockSpec((BR, D), lambda i: (i, 0)),
    ],
    out_specs=pl.BlockSpec((BR, D), lambda i: (i, 0)),
    out_shape=jax.ShapeDtypeStruct((N, D), jnp.float32),
)


def kernel(x, edge_index, Wl1, Wr1, b1, Wl2, Wr2, b2):
    ei = edge_index.astype(jnp.int32)
    pad = EP - E
    ar = jnp.arange(pad, dtype=jnp.int32)
    src = jnp.concatenate([ei[0], ar % N])
    dst = jnp.concatenate([ei[1], N + (ar % (NP - N))])
    eidx = jnp.stack([src.reshape(NW, KN, KB), dst.reshape(NW, KN, KB)],
                     axis=2)
    znd = jnp.zeros((NP, D), jnp.float32)
    zn = jnp.zeros((NP,), jnp.float32)
    wls = jnp.stack([Wl1, Wl2])
    wrs = jnp.stack([Wr1, Wr2])
    bs = jnp.stack([b1.reshape(1, D), b2.reshape(1, D)])

    def layer(i, h):
        wl = lax.dynamic_index_in_dim(wls, i, keepdims=False)
        wr = lax.dynamic_index_in_dim(wrs, i, keepdims=False)
        b = lax.dynamic_index_in_dim(bs, i, keepdims=False)
        y, r = _pre(h, wl, wr, b)
        agg, cnt = _sc_segsum(y, eidx, znd, zn)
        return _fin(agg, cnt.T, r)

    return lax.fori_loop(0, 2, layer, x)
